# baseline (speedup 1.0000x reference)
import jax
import jax.numpy as jnp
from jax import lax
from jax.experimental import pallas as pl
from jax.experimental.pallas import tpu as pltpu
from jax.experimental.pallas import tpu_sc as plsc

B, N, F, D, TOPK = 32, 1024, 64, 64, 32
BN_EPS = 1e-5
NEG_INF = float("-inf")
NODES_PER_W = 32
EPW = NODES_PER_W * TOPK


def _topk_mask_kernel(emb_ref, w1_ref, w2_ref, idx_ref, mask_ref, w21_ref, cos_ref):
    w = emb_ref[:]
    g = jax.lax.dot_general(
        w, w, (((1,), (1,)), ((), ())), preferred_element_type=jnp.float32
    )
    sq = jnp.sum(w * w, axis=1, keepdims=True)
    nrm = jnp.sqrt(sq)
    row_i = jax.lax.broadcasted_iota(jnp.int32, (N, N), 0)
    col_i = jax.lax.broadcasted_iota(jnp.int32, (N, N), 1)
    eye = jnp.where(row_i == col_i, 1.0, 0.0)
    nrm_row = jax.lax.dot_general(
        nrm, eye, (((0,), (0,)), ((), ())),
        precision=jax.lax.Precision.HIGHEST,
        preferred_element_type=jnp.float32,
    )
    cos_ref[:] = g / (nrm * nrm_row)

    lane = jax.lax.broadcasted_iota(jnp.int32, (N, N), 1)
    for k in range(TOPK):
        c = cos_ref[:]
        amax = jnp.argmax(c, axis=1, keepdims=True).astype(jnp.int32)
        idx_ref[:, k : k + 1] = amax
        cos_ref[:] = jnp.where(lane == amax, NEG_INF, c)
    mask_ref[:] = jnp.where(cos_ref[:] == NEG_INF, 0.0, NEG_INF)

    w21 = jax.lax.dot_general(
        w2_ref[:].astype(jnp.bfloat16), w1_ref[:].astype(jnp.bfloat16),
        (((1,), (0,)), ((), ())), preferred_element_type=jnp.float32,
    )
    w21_ref[:] = w21.astype(jnp.bfloat16)


def _feat_kernel(data_ref, emb_ref, linw_ref, atti_ref, attj_ref,
                 xt_ref, si_ref, sj_ref):
    d = data_ref[0]
    xt = jax.lax.dot_general(
        d, linw_ref[:], (((0,), (1,)), ((), ())),
        preferred_element_type=jnp.float32,
    )
    emb = emb_ref[:]
    s_i = (
        jnp.dot(xt, atti_ref[:D, :], preferred_element_type=jnp.float32)
        + jnp.dot(emb, atti_ref[D:, :], preferred_element_type=jnp.float32)
    )
    s_j = (
        jnp.dot(xt, attj_ref[:D, :], preferred_element_type=jnp.float32)
        + jnp.dot(emb, attj_ref[D:, :], preferred_element_type=jnp.float32)
    )
    xt_ref[0] = xt
    si_ref[0] = s_i
    sj_ref[0] = s_j


def _sc_agg_kernel(xt_hbm, si_hbm, sj_hbm, idx_hbm, out_hbm,
                   idxflat_v, gidx_v, rows_v, sjb_v, sib_v, attn_v, out_v, sem):
    cid = lax.axis_index("c")
    sid = lax.axis_index("s")
    wid = sid * 2 + cid
    n0 = wid * NODES_PER_W

    pltpu.sync_copy(idx_hbm.at[pl.ds(n0 * TOPK, EPW)], idxflat_v)

    lane = lax.iota(jnp.int32, 16)

    def batch_body(b, _):
        base = b * N

        def upd(c, _):
            seg = idxflat_v[pl.ds(c * 16, 16)]
            gidx_v[pl.ds(c * 16, 16)] = seg + base
            return 0
        lax.fori_loop(0, EPW // 16, upd, 0, unroll=4)

        pltpu.sync_copy(sj_hbm.at[pl.ds(base, N)], sjb_v)
        pltpu.sync_copy(si_hbm.at[pl.ds(base + n0, NODES_PER_W)], sib_v)

        copies = []
        for c in range(8):
            copies.append(pltpu.async_copy(
                xt_hbm.at[gidx_v.at[pl.ds(c * 128, 128)]],
                rows_v.at[pl.ds(c * 128, 128)], sem))
        for cp in copies:
            cp.wait()

        for g in range(NODES_PER_W // 16):
            si = sib_v[pl.ds(g * 16, 16)]
            alphas = []
            for k in range(TOPK):
                eidx = lane * TOPK + (g * 16 * TOPK + k)
                jl = plsc.load_gather(idxflat_v, [eidx])
                sj = plsc.load_gather(sjb_v, [jl])
                a = si + sj
                a = jnp.where(a >= 0.0, a, 0.2 * a)
                alphas.append(a)
            m = alphas[0]
            for k in range(1, TOPK):
                m = jnp.maximum(m, alphas[k])
            exps = [jnp.exp(a - m) for a in alphas]
            den = exps[0]
            for k in range(1, TOPK):
                den = den + exps[k]
            for k in range(TOPK):
                attn_v[pl.ds(k * NODES_PER_W + g * 16, 16)] = exps[k] / den

        def node_body(nl, _):
            accs = [jnp.zeros((16,), jnp.float32) for _ in range(D // 16)]
            for k in range(TOPK):
                w_b = plsc.load_gather(attn_v, [jnp.zeros((16,), jnp.int32)
                                                + (k * NODES_PER_W + nl)])
                for g4 in range(D // 16):
                    row = rows_v[nl * TOPK + k, pl.ds(g4 * 16, 16)]
                    accs[g4] = accs[g4] + w_b * row
            for g4 in range(D // 16):
                out_v[nl, pl.ds(g4 * 16, 16)] = accs[g4]
            return 0
        lax.fori_loop(0, NODES_PER_W, node_body, 0)

        pltpu.sync_copy(out_v, out_hbm.at[pl.ds(base + n0, NODES_PER_W)])
        return 0

    lax.fori_loop(0, B, batch_body, 0)


def _post_kernel(agg_ref, emb_ref, bias_ref, gamma_ref, beta_ref,
                 w21_ref, b2_ref, out_ref):
    agg = agg_ref[0] + bias_ref[:]
    agg = gamma_ref[:] * (agg * (1.0 / (1.0 + BN_EPS) ** 0.5)) + beta_ref[:]
    gcn = jnp.maximum(agg, 0.0)
    p = (gcn * emb_ref[:]).astype(jnp.bfloat16)
    o = jax.lax.dot_general(
        p, w21_ref[:], (((0,), (1,)), ((), ())), preferred_element_type=jnp.float32
    )
    out_ref[0] = jax.nn.sigmoid(o + b2_ref[:])


def kernel(data, emb_table, lin_W, att_i, att_j, gnn_bias, bn_gamma, bn_beta, W1, W2, b2):
    topk_idx, mask, w21 = pl.pallas_call(
        _topk_mask_kernel,
        out_shape=(
            jax.ShapeDtypeStruct((N, TOPK), jnp.int32),
            jax.ShapeDtypeStruct((N, N), jnp.float32),
            jax.ShapeDtypeStruct((N, N), jnp.bfloat16),
        ),
        scratch_shapes=[pltpu.VMEM((N, N), jnp.float32)],
    )(jax.lax.stop_gradient(emb_table), W1, W2)

    full = lambda shape: pl.BlockSpec(shape, lambda b: (0,) * len(shape))
    xt, s_i, s_j = pl.pallas_call(
        _feat_kernel,
        grid=(B,),
        in_specs=[
            pl.BlockSpec((1, F, N), lambda b: (b, 0, 0)),
            full((N, D)),
            full((D, F)),
            full((2 * D, 1)),
            full((2 * D, 1)),
        ],
        out_specs=(
            pl.BlockSpec((1, N, D), lambda b: (b, 0, 0)),
            pl.BlockSpec((1, N, 1), lambda b: (b, 0, 0)),
            pl.BlockSpec((1, N, 1), lambda b: (b, 0, 0)),
        ),
        out_shape=(
            jax.ShapeDtypeStruct((B, N, D), jnp.float32),
            jax.ShapeDtypeStruct((B, N, 1), jnp.float32),
            jax.ShapeDtypeStruct((B, N, 1), jnp.float32),
        ),
    )(
        data, emb_table, lin_W,
        att_i.reshape(2 * D, 1), att_j.reshape(2 * D, 1),
    )

    mesh = plsc.VectorSubcoreMesh(core_axis_name="c", subcore_axis_name="s")
    agg = pl.kernel(
        _sc_agg_kernel,
        mesh=mesh,
        compiler_params=pltpu.CompilerParams(
            needs_layout_passes=False, use_tc_tiling_on_sc=False),
        out_type=jax.ShapeDtypeStruct((B * N, D), jnp.float32),
        scratch_types=[
            pltpu.VMEM((EPW,), jnp.int32),
            pltpu.VMEM((EPW,), jnp.int32),
            pltpu.VMEM((EPW, D), jnp.float32),
            pltpu.VMEM((N,), jnp.float32),
            pltpu.VMEM((NODES_PER_W,), jnp.float32),
            pltpu.VMEM((EPW,), jnp.float32),
            pltpu.VMEM((NODES_PER_W, D), jnp.float32),
            pltpu.SemaphoreType.DMA,
        ],
    )(
        xt.reshape(B * N, D),
        s_i.reshape(B * N),
        s_j.reshape(B * N),
        topk_idx.reshape(N * TOPK),
    )

    out = pl.pallas_call(
        _post_kernel,
        grid=(B,),
        in_specs=[
            pl.BlockSpec((1, N, D), lambda b: (b, 0, 0)),
            full((N, D)),
            full((1, D)),
            full((1, D)),
            full((1, D)),
            full((N, N)),
            full((1, N)),
        ],
        out_specs=pl.BlockSpec((1, D, N), lambda b: (b, 0, 0)),
        out_shape=jax.ShapeDtypeStruct((B, D, N), jnp.float32),
    )(
        agg.reshape(B, N, D),
        emb_table,
        gnn_bias.reshape(1, D),
        bn_gamma.reshape(1, D),
        bn_beta.reshape(1, D),
        w21,
        b2.reshape(1, N),
    )
    return (out, emb_table, topk_idx)
